# baseline (speedup 1.0000x reference)
import functools

import jax
import jax.numpy as jnp
from jax import lax
from jax.experimental import pallas as pl
from jax.experimental.pallas import tpu as pltpu
from jax.experimental.pallas import tpu_sc as plsc

N = 10000
NP = 10240
E = 320000
F_IN = 128
H = 512
G = 64
NC = 10

NW = 32
NPW = NP // NW
NBLK = 16
CH = 16
OFFW = NPW + 16
WIN = 2048
E_ELL = E + 15 * N + 3600

RB = 512
NRED = 25
RRB = 400

@functools.lru_cache(maxsize=None)
def _mesh():
    return plsc.VectorSubcoreMesh(
        core_axis_name="c", subcore_axis_name="s", num_cores=2,
        num_subcores=16)


@functools.lru_cache(maxsize=None)
def _make_agg():
    @functools.partial(
        pl.kernel,
        out_type=jax.ShapeDtypeStruct((NP, H), jnp.float32),
        mesh=_mesh(),
        scratch_types=[
            pltpu.VMEM((OFFW,), jnp.int32),
            pltpu.VMEM((OFFW,), jnp.float32),
            pltpu.VMEM((NBLK, H), jnp.float32),
            pltpu.VMEM((NBLK, H), jnp.float32),
            pltpu.VMEM((NBLK, H), jnp.float32),
            pltpu.VMEM((WIN,), jnp.int32),
            pltpu.VMEM((CH,), jnp.int32),
            pltpu.VMEM((4, CH, H), jnp.float32),
            pltpu.VMEM((H,), jnp.float32),
            pltpu.SemaphoreType.DMA,
            pltpu.SemaphoreType.DMA,
            pltpu.SemaphoreType.DMA,
            pltpu.SemaphoreType.DMA,
        ],
    )
    def agg(h_hbm, gam_hbm, bet_hbm, base_hbm, off_hbm, inv_hbm, src_hbm,
            y_hbm, offv, invv, gv, bv, resv, idxw, idxf, rows4, accv,
            sem0, sem1, sem2, sem3):
        sems = (sem0, sem1, sem2, sem3)
        wid = lax.axis_index("s") * 2 + lax.axis_index("c")
        base = pl.multiple_of(wid * NPW, NPW)
        pltpu.sync_copy(off_hbm.at[pl.ds(base, OFFW)], offv)
        pltpu.sync_copy(inv_hbm.at[pl.ds(base, OFFW)], invv)

        def blk_body(b, carry):
            d0 = pl.multiple_of(base + b * NBLK, NBLK)
            pltpu.sync_copy(gam_hbm.at[pl.ds(d0, NBLK)], gv)
            pltpu.sync_copy(bet_hbm.at[pl.ds(d0, NBLK)], bv)
            pltpu.sync_copy(base_hbm.at[pl.ds(d0, NBLK)], resv)
            bstart = offv[pl.ds(b * NBLK, 16)][0]
            wbase = pl.multiple_of((bstart // CH) * CH, CH)
            pltpu.sync_copy(src_hbm.at[pl.ds(wbase, WIN)], idxw)

            def node_body(j, carry2):
                n = b * NBLK + j
                ovec = offv[pl.ds(n, 16)]
                s = ovec[0]
                e = ovec[1]
                inv = invv[pl.ds(n, 16)][0]
                a0 = pl.multiple_of((s // CH) * CH, CH)
                nch = (e - a0 + CH - 1) // CH
                for c in range(H // 16):
                    accv[pl.ds(c * 16, 16)] = jnp.zeros((16,), jnp.float32)

                def idx_at(k):
                    woff = pl.multiple_of(a0 + k * CH - wbase, CH)
                    return idxw.at[pl.ds(woff, CH)]

                def issue_w(k, i):
                    pltpu.async_copy(h_hbm.at[idx_at(k)], rows4.at[i], sems[i])

                def compute(k, idxv, rowsv, sem):
                    pltpu.make_async_copy(h_hbm.at[idxv], rowsv, sem).wait()
                    kb = a0 + k * CH
                    for half in range(2):
                        c0 = half * (H // 2)
                        gr = [gv[j, pl.ds(c0 + c * 16, 16)] for c in range(16)]
                        br = [bv[j, pl.ds(c0 + c * 16, 16)] for c in range(16)]
                        ac = [accv[pl.ds(c0 + c * 16, 16)] for c in range(16)]

                        for r in range(CH):
                            gpos = kb + r
                            valid = jnp.logical_and(gpos >= s, gpos < e)
                            for c in range(16):
                                m = jnp.maximum(
                                    gr[c] * rowsv[r, pl.ds(c0 + c * 16, 16)]
                                    + br[c], 0.0)
                                ac[c] = ac[c] + jnp.where(valid, m, 0.0)
                        for c in range(16):
                            accv[pl.ds(c0 + c * 16, 16)] = ac[c]

                fits = e <= wbase + WIN

                @pl.when(fits)
                def _():
                    for i in range(3):
                        @pl.when(i < nch)
                        def _():
                            issue_w(i, i)

                    def ch_body(k, carry3):
                        ph = lax.rem(k, 4)
                        for i in range(4):
                            @pl.when(ph == i)
                            def _():
                                compute(k, idx_at(k), rows4.at[i], sems[i])

                                @pl.when(k + 3 < nch)
                                def _():
                                    issue_w(k + 3, (i + 3) % 4)

                        return carry3

                    lax.fori_loop(0, nch, ch_body, 0)

                @pl.when(jnp.logical_not(fits))
                def _():
                    def ch_body_f(k, carry3):
                        pltpu.sync_copy(
                            src_hbm.at[
                                pl.ds(pl.multiple_of(a0 + k * CH, CH), CH)],
                            idxf)
                        pltpu.async_copy(h_hbm.at[idxf], rows4.at[0], sem0)
                        compute(k, idxf, rows4.at[0], sem0)
                        return carry3

                    lax.fori_loop(0, nch, ch_body_f, 0)

                for c in range(H // 16):
                    resv[j, pl.ds(c * 16, 16)] = (
                        resv[j, pl.ds(c * 16, 16)]
                        + accv[pl.ds(c * 16, 16)] * inv)
                return carry2

            lax.fori_loop(0, NBLK, node_body, 0)
            pltpu.sync_copy(resv, y_hbm.at[pl.ds(d0, NBLK)])
            return carry

        lax.fori_loop(0, NPW // NBLK, blk_body, 0)

    return agg


RCH = 16


@functools.lru_cache(maxsize=None)
def _make_readout():
    @functools.partial(
        pl.kernel,
        out_type=jax.ShapeDtypeStruct((G, 2 * H), jnp.float32),
        mesh=_mesh(),
        scratch_types=[
            pltpu.VMEM((80,), jnp.int32),
            pltpu.VMEM((80,), jnp.float32),
            pltpu.VMEM((RCH, H), jnp.float32),
            pltpu.VMEM((H,), jnp.float32),
            pltpu.VMEM((H,), jnp.float32),
            pltpu.VMEM((1, 2 * H), jnp.float32),
        ],
    )
    def _readout(x_hbm, boff_hbm, binv_hbm, r_hbm, offv, binvv, rowsv, sumv,
                 maxv, outv):
        wid = lax.axis_index("s") * 2 + lax.axis_index("c")
        pltpu.sync_copy(boff_hbm.at[pl.ds(0, 80)], offv)
        pltpu.sync_copy(binv_hbm.at[pl.ds(0, 80)], binvv)
        for t in range(2):
            g = wid * 2 + t
            ovec = offv[pl.ds(g, 16)]
            s = ovec[0]
            e = ovec[1]
            cnt = e - s
            a0 = pl.multiple_of((s // 8) * 8, 8)
            neg = jnp.full((16,), -jnp.inf, jnp.float32)
            for c in range(H // 16):
                sumv[pl.ds(c * 16, 16)] = jnp.zeros((16,), jnp.float32)
                maxv[pl.ds(c * 16, 16)] = neg
            nch = (e - a0 + RCH - 1) // RCH

            def ch_body(k, carry):
                start = pl.multiple_of(a0 + k * RCH, 8)
                pltpu.sync_copy(x_hbm.at[pl.ds(start, RCH)], rowsv)
                for half in range(2):
                    c0 = half * (H // 2)
                    sm = [sumv[pl.ds(c0 + c * 16, 16)] for c in range(16)]
                    mx = [maxv[pl.ds(c0 + c * 16, 16)] for c in range(16)]

                    def row_body(r, sv):
                        sm2, mx2 = sv
                        gidx = a0 + k * RCH + r
                        valid = jnp.logical_and(gidx >= s, gidx < e)
                        sm3, mx3 = [], []
                        for c in range(16):
                            raw = rowsv[r, pl.ds(c0 + c * 16, 16)]
                            row = jnp.where(
                                raw > 0.0, raw,
                                jnp.exp(jnp.minimum(raw, 0.0)) - 1.0)
                            sm3.append(sm2[c] + jnp.where(valid, row, 0.0))
                            mx3.append(jnp.maximum(
                                mx2[c], jnp.where(valid, row, -jnp.inf)))
                        return tuple(sm3), tuple(mx3)

                    sm, mx = lax.fori_loop(0, RCH, row_body,
                                           (tuple(sm), tuple(mx)))
                    for c in range(16):
                        sumv[pl.ds(c0 + c * 16, 16)] = sm[c]
                        maxv[pl.ds(c0 + c * 16, 16)] = mx[c]
                return carry

            lax.fori_loop(0, nch, ch_body, 0)
            inv = binvv[pl.ds(g, 16)][0]
            for c in range(H // 16):
                outv[0, pl.ds(c * 16, 16)] = sumv[pl.ds(c * 16, 16)] * inv
                outv[0, pl.ds(H + c * 16, 16)] = maxv[pl.ds(c * 16, 16)]
            pltpu.sync_copy(outv, r_hbm.at[pl.ds(g, 1)])

    return _readout


def _mm_body_plain(x_ref, w_ref, bf_ref, h_ref, gam_ref, bet_ref, out_ref):
    _mm_core(x_ref[...], w_ref, bf_ref, h_ref, gam_ref, bet_ref, out_ref)


def _mm_body_elu(x_ref, w_ref, bf_ref, h_ref, gam_ref, bet_ref, out_ref):
    xb = x_ref[...]
    xb = jnp.where(xb > 0.0, xb, jnp.exp(jnp.minimum(xb, 0.0)) - 1.0)
    _mm_core(xb, w_ref, bf_ref, h_ref, gam_ref, bet_ref, out_ref)


def _mm_body_bn(x_ref, w_ref, bf_ref, mv_ref, g_ref, b_ref,
                h_ref, gam_ref, bet_ref, out_ref):
    xb = x_ref[...]
    mean = mv_ref[0, :]
    var = mv_ref[1, :]
    xn = (xb - mean[None, :]) * lax.rsqrt(var + 1e-5)[None, :]
    xn = xn * g_ref[0, :][None, :] + b_ref[0, :][None, :]
    xn = jnp.where(xn > 0.0, xn, jnp.exp(jnp.minimum(xn, 0.0)) - 1.0)
    rows = lax.broadcasted_iota(jnp.int32, xb.shape, 0) + pl.program_id(0) * RB
    xb = jnp.where(rows < N, xn, 0.0)
    _mm_core(xb, w_ref, bf_ref, h_ref, gam_ref, bet_ref, out_ref)


def _mm_core(xb, w_ref, bf_ref, h_ref, gam_ref, bet_ref, out_ref):
    y = jnp.dot(xb, w_ref[...], preferred_element_type=jnp.float32)
    h_ref[...] = y[:, :H]
    film = y[:, H:3 * H] + bf_ref[0, :][None, :]
    bet_ref[...] = film[:, :H]
    gam_ref[...] = film[:, H:]
    out_ref[...] = jnp.maximum(
        y[:, 5 * H:6 * H] * y[:, 3 * H:4 * H] + y[:, 4 * H:5 * H], 0.0)


def _layer_mm(xin, wcat, bfilm, bn=None):
    din = xin.shape[1]
    outs = [jax.ShapeDtypeStruct((NP, H), jnp.float32)] * 4
    full = lambda a: pl.BlockSpec(a.shape, lambda i: (0,) * a.ndim)
    in_specs = [
        pl.BlockSpec((RB, din), lambda i: (i, 0)),
        full(wcat),
        full(bfilm),
    ]
    args = [xin, wcat, bfilm]
    if bn is None:
        body = _mm_body_plain
    elif bn == "elu":
        body = _mm_body_elu
    else:
        body = _mm_body_bn
        mv, gg, gb = bn
        in_specs += [full(mv), full(gg), full(gb)]
        args += [mv, gg, gb]
    return pl.pallas_call(
        body,
        grid=(NP // RB,),
        in_specs=in_specs,
        out_specs=[pl.BlockSpec((RB, H), lambda i: (i, 0))] * 4,
        out_shape=outs,
    )(*args)


def _red_body(y_ref, mv_ref):
    i = pl.program_id(0)

    @pl.when(i == 0)
    def _():
        mv_ref[...] = jnp.zeros_like(mv_ref)

    yb = y_ref[...]
    mv_ref[0, :] += jnp.sum(yb, axis=0)
    mv_ref[1, :] += jnp.sum(yb * yb, axis=0)

    @pl.when(i == NRED - 1)
    def _():
        m = mv_ref[0, :] * (1.0 / N)
        v = mv_ref[1, :] * (1.0 / N) - m * m
        mv_ref[0, :] = m
        mv_ref[1, :] = v


def _col_meanvar(y):
    return pl.pallas_call(
        _red_body,
        grid=(NRED,),
        in_specs=[pl.BlockSpec((RRB, H), lambda i: (i, 0))],
        out_specs=pl.BlockSpec((8, H), lambda i: (0, 0)),
        out_shape=jax.ShapeDtypeStruct((8, H), jnp.float32),
    )(y)


def _head_body(r_ref, w1_ref, b1_ref, g_ref, b_ref, w2_ref, b2_ref, o_ref):
    z = jnp.dot(r_ref[...], w1_ref[...], preferred_element_type=jnp.float32)
    z = z + b1_ref[0, :][None, :]
    m = jnp.mean(z, axis=0)
    v = jnp.mean(z * z, axis=0) - m * m
    zn = (z - m[None, :]) * lax.rsqrt(v + 1e-5)[None, :]
    zn = zn * g_ref[0, :][None, :] + b_ref[0, :][None, :]
    ze = jnp.where(zn > 0.0, zn, jnp.exp(jnp.minimum(zn, 0.0)) - 1.0)
    o_ref[...] = jnp.dot(ze, w2_ref[...],
                         preferred_element_type=jnp.float32) + b2_ref[0, :][None, :]


def _head(r, w1, b1, g, b, w2p, b2p):
    full = lambda a: pl.BlockSpec(a.shape, lambda: (0,) * a.ndim)
    return pl.pallas_call(
        _head_body,
        in_specs=[full(r), full(w1), full(b1), full(g), full(b),
                  full(w2p), full(b2p)],
        out_specs=full(jax.ShapeDtypeStruct((G, 128), jnp.float32)),
        out_shape=jax.ShapeDtypeStruct((G, 128), jnp.float32),
    )(r, w1, b1, g, b, w2p, b2p)


def kernel(x, edge_index, edge_attr, batch,
           W1_lin, W1_film, b1_film, W1_skip, W1_fs,
           W2_lin, W2_film, b2_film, W2_skip, W2_fs,
           W3_lin, W3_film, b3_film, W3_skip, W3_fs,
           gbn_g, gbn_b, bn_g, bn_b, lin1_W, lin1_b, cls_W, cls_b):
    src, dst = edge_index[0], edge_index[1]

    packed = lax.sort(dst * 16384 + src)
    src_s = packed & 16383
    row_off = jnp.searchsorted(
        packed, jnp.arange(NP + 1, dtype=jnp.int32) * 16384).astype(jnp.int32)
    deg = row_off[1:] - row_off[:-1]
    off_p = jnp.pad(row_off, (0, (NW - 1) * NPW + OFFW - (NP + 1)),
                    mode="edge")
    deg_p = jnp.pad(deg, (0, (NW - 1) * NPW + OFFW - NP))
    invdeg_p = 1.0 / jnp.maximum(deg_p.astype(jnp.float32), 1.0)
    src_pad = jnp.pad(src_s, (0, WIN + 512))
    b_off = jnp.searchsorted(
        batch, jnp.arange(G + 1, dtype=jnp.int32)).astype(jnp.int32)
    b_off_p = jnp.pad(b_off, (0, 80 - (G + 1)), mode="edge")
    binv_p = 1.0 / jnp.maximum(
        (b_off_p[1:] - b_off_p[:-1]).astype(jnp.float32), 1.0)
    binv_p = jnp.pad(binv_p, (0, 80 - binv_p.shape[0]))

    xp = jnp.pad(x, ((0, NP - N), (0, 0)))
    wcat1 = jnp.concatenate([W1_lin, W1_film, W1_skip, W1_fs], axis=1)
    wcat2 = jnp.concatenate([W2_lin, W2_film, W2_skip, W2_fs], axis=1)
    wcat3 = jnp.concatenate([W3_lin, W3_film, W3_skip, W3_fs], axis=1)
    bf1 = b1_film.reshape(1, 2 * H)
    bf2 = b2_film.reshape(1, 2 * H)
    bf3 = b3_film.reshape(1, 2 * H)

    h, gam, bet, out = _layer_mm(xp, wcat1, bf1)
    y1 = _make_agg()(h, gam, bet, out, off_p, invdeg_p, src_pad)
    mv = _col_meanvar(y1)

    h, gam, bet, out = _layer_mm(
        y1, wcat2, bf2, bn=(mv, gbn_g.reshape(1, H), gbn_b.reshape(1, H)))
    x2 = _make_agg()(h, gam, bet, out, off_p, invdeg_p, src_pad)

    h, gam, bet, out = _layer_mm(x2, wcat3, bf3, bn="elu")
    x3 = _make_agg()(h, gam, bet, out, off_p, invdeg_p, src_pad)

    r = _make_readout()(x3, b_off_p, binv_p)
    w2p = jnp.pad(cls_W, ((0, 0), (0, 128 - NC)))
    b2p = jnp.pad(cls_b, (0, 128 - NC)).reshape(1, 128)
    logits = _head(r, lin1_W, lin1_b.reshape(1, H),
                   bn_g.reshape(1, H), bn_b.reshape(1, H), w2p, b2p)
    return logits[:, :NC]


# speedup vs baseline: 1.8327x; 1.8327x over previous
import functools

import jax
import jax.numpy as jnp
from jax import lax
from jax.experimental import pallas as pl
from jax.experimental.pallas import tpu as pltpu
from jax.experimental.pallas import tpu_sc as plsc

N = 10000
NP = 10240
E = 320000
F_IN = 128
H = 512
G = 64
NC = 10

NW = 32
NPW = NP // NW
NBLK = 16
CH = 16
OFFW = NPW + 16
WIN = 2048
E_ELL = E + 15 * N + 3600

RB = 512
NRED = 25
RRB = 400

@functools.lru_cache(maxsize=None)
def _mesh():
    return plsc.VectorSubcoreMesh(
        core_axis_name="c", subcore_axis_name="s", num_cores=2,
        num_subcores=16)


@functools.lru_cache(maxsize=None)
def _make_agg():
    @functools.partial(
        pl.kernel,
        out_type=jax.ShapeDtypeStruct((NP, H), jnp.float32),
        mesh=_mesh(),
        scratch_types=[
            pltpu.VMEM((OFFW,), jnp.int32),
            pltpu.VMEM((OFFW,), jnp.float32),
            pltpu.VMEM((NBLK, H), jnp.float32),
            pltpu.VMEM((NBLK, H), jnp.float32),
            pltpu.VMEM((NBLK, H), jnp.float32),
            pltpu.VMEM((WIN,), jnp.int32),
            pltpu.VMEM((CH,), jnp.int32),
            pltpu.VMEM((4, CH, H), jnp.float32),
            pltpu.VMEM((H,), jnp.float32),
            pltpu.SemaphoreType.DMA,
            pltpu.SemaphoreType.DMA,
            pltpu.SemaphoreType.DMA,
            pltpu.SemaphoreType.DMA,
        ],
    )
    def agg(h_hbm, gam_hbm, bet_hbm, base_hbm, off_hbm, inv_hbm, src_hbm,
            y_hbm, offv, invv, gv, bv, resv, idxw, idxf, rows4, accv,
            sem0, sem1, sem2, sem3):
        sems = (sem0, sem1, sem2, sem3)
        wid = lax.axis_index("s") * 2 + lax.axis_index("c")
        base = pl.multiple_of(wid * NPW, NPW)
        pltpu.sync_copy(off_hbm.at[pl.ds(base, OFFW)], offv)
        pltpu.sync_copy(inv_hbm.at[pl.ds(base, OFFW)], invv)

        def blk_body(b, carry):
            d0 = pl.multiple_of(base + b * NBLK, NBLK)
            pltpu.sync_copy(gam_hbm.at[pl.ds(d0, NBLK)], gv)
            pltpu.sync_copy(bet_hbm.at[pl.ds(d0, NBLK)], bv)
            pltpu.sync_copy(base_hbm.at[pl.ds(d0, NBLK)], resv)
            bstart = offv[pl.ds(b * NBLK, 16)][0]
            wbase = pl.multiple_of((bstart // CH) * CH, CH)
            pltpu.sync_copy(src_hbm.at[pl.ds(wbase, WIN)], idxw)

            def node_body(j, carry2):
                n = b * NBLK + j
                ovec = offv[pl.ds(n, 16)]
                s = ovec[0]
                e = ovec[1]
                inv = invv[pl.ds(n, 16)][0]
                a0 = pl.multiple_of((s // CH) * CH, CH)
                nch = (e - a0 + CH - 1) // CH
                for c in range(H // 16):
                    accv[pl.ds(c * 16, 16)] = jnp.zeros((16,), jnp.float32)

                def idx_at(k):
                    woff = pl.multiple_of(a0 + k * CH - wbase, CH)
                    return idxw.at[pl.ds(woff, CH)]

                def issue_w(k, i):
                    pltpu.async_copy(h_hbm.at[idx_at(k)], rows4.at[i], sems[i])

                def compute(k, idxv, rowsv, sem):
                    pltpu.make_async_copy(h_hbm.at[idxv], rowsv, sem).wait()
                    kb = a0 + k * CH
                    for half in range(2):
                        c0 = half * (H // 2)
                        gr = [gv[j, pl.ds(c0 + c * 16, 16)] for c in range(16)]
                        br = [bv[j, pl.ds(c0 + c * 16, 16)] for c in range(16)]
                        ac = [accv[pl.ds(c0 + c * 16, 16)] for c in range(16)]

                        def row_body(r, acc):
                            gpos = kb + r
                            valid = jnp.logical_and(gpos >= s, gpos < e)
                            out = []
                            for c in range(16):
                                m = jnp.maximum(
                                    gr[c] * rowsv[r, pl.ds(c0 + c * 16, 16)]
                                    + br[c], 0.0)
                                out.append(
                                    acc[c] + jnp.where(valid, m, 0.0))
                            return tuple(out)

                        ac = lax.fori_loop(0, CH, row_body, tuple(ac))
                        ac = list(ac)
                        for c in range(16):
                            accv[pl.ds(c0 + c * 16, 16)] = ac[c]

                fits = e <= wbase + WIN

                @pl.when(fits)
                def _():
                    for i in range(3):
                        @pl.when(i < nch)
                        def _():
                            issue_w(i, i)

                    def ch_body(k, carry3):
                        ph = lax.rem(k, 4)
                        for i in range(4):
                            @pl.when(ph == i)
                            def _():
                                compute(k, idx_at(k), rows4.at[i], sems[i])

                                @pl.when(k + 3 < nch)
                                def _():
                                    issue_w(k + 3, (i + 3) % 4)

                        return carry3

                    lax.fori_loop(0, nch, ch_body, 0)

                @pl.when(jnp.logical_not(fits))
                def _():
                    def ch_body_f(k, carry3):
                        pltpu.sync_copy(
                            src_hbm.at[
                                pl.ds(pl.multiple_of(a0 + k * CH, CH), CH)],
                            idxf)
                        pltpu.async_copy(h_hbm.at[idxf], rows4.at[0], sem0)
                        compute(k, idxf, rows4.at[0], sem0)
                        return carry3

                    lax.fori_loop(0, nch, ch_body_f, 0)

                for c in range(H // 16):
                    resv[j, pl.ds(c * 16, 16)] = (
                        resv[j, pl.ds(c * 16, 16)]
                        + accv[pl.ds(c * 16, 16)] * inv)
                return carry2

            lax.fori_loop(0, NBLK, node_body, 0)
            pltpu.sync_copy(resv, y_hbm.at[pl.ds(d0, NBLK)])
            return carry

        lax.fori_loop(0, NPW // NBLK, blk_body, 0)

    return agg


RCH = 16


@functools.lru_cache(maxsize=None)
def _make_readout():
    @functools.partial(
        pl.kernel,
        out_type=jax.ShapeDtypeStruct((G, 2 * H), jnp.float32),
        mesh=_mesh(),
        scratch_types=[
            pltpu.VMEM((80,), jnp.int32),
            pltpu.VMEM((80,), jnp.float32),
            pltpu.VMEM((RCH, H), jnp.float32),
            pltpu.VMEM((H,), jnp.float32),
            pltpu.VMEM((H,), jnp.float32),
            pltpu.VMEM((1, 2 * H), jnp.float32),
        ],
    )
    def _readout(x_hbm, boff_hbm, binv_hbm, r_hbm, offv, binvv, rowsv, sumv,
                 maxv, outv):
        wid = lax.axis_index("s") * 2 + lax.axis_index("c")
        pltpu.sync_copy(boff_hbm.at[pl.ds(0, 80)], offv)
        pltpu.sync_copy(binv_hbm.at[pl.ds(0, 80)], binvv)
        for t in range(2):
            g = wid * 2 + t
            ovec = offv[pl.ds(g, 16)]
            s = ovec[0]
            e = ovec[1]
            cnt = e - s
            a0 = pl.multiple_of((s // 8) * 8, 8)
            neg = jnp.full((16,), -jnp.inf, jnp.float32)
            for c in range(H // 16):
                sumv[pl.ds(c * 16, 16)] = jnp.zeros((16,), jnp.float32)
                maxv[pl.ds(c * 16, 16)] = neg
            nch = (e - a0 + RCH - 1) // RCH

            def ch_body(k, carry):
                start = pl.multiple_of(a0 + k * RCH, 8)
                pltpu.sync_copy(x_hbm.at[pl.ds(start, RCH)], rowsv)
                for half in range(2):
                    c0 = half * (H // 2)
                    sm = [sumv[pl.ds(c0 + c * 16, 16)] for c in range(16)]
                    mx = [maxv[pl.ds(c0 + c * 16, 16)] for c in range(16)]

                    def row_body(r, sv):
                        sm2, mx2 = sv
                        gidx = a0 + k * RCH + r
                        valid = jnp.logical_and(gidx >= s, gidx < e)
                        sm3, mx3 = [], []
                        for c in range(16):
                            raw = rowsv[r, pl.ds(c0 + c * 16, 16)]
                            row = jnp.where(
                                raw > 0.0, raw,
                                jnp.exp(jnp.minimum(raw, 0.0)) - 1.0)
                            sm3.append(sm2[c] + jnp.where(valid, row, 0.0))
                            mx3.append(jnp.maximum(
                                mx2[c], jnp.where(valid, row, -jnp.inf)))
                        return tuple(sm3), tuple(mx3)

                    sm, mx = lax.fori_loop(0, RCH, row_body,
                                           (tuple(sm), tuple(mx)))
                    for c in range(16):
                        sumv[pl.ds(c0 + c * 16, 16)] = sm[c]
                        maxv[pl.ds(c0 + c * 16, 16)] = mx[c]
                return carry

            lax.fori_loop(0, nch, ch_body, 0)
            inv = binvv[pl.ds(g, 16)][0]
            for c in range(H // 16):
                outv[0, pl.ds(c * 16, 16)] = sumv[pl.ds(c * 16, 16)] * inv
                outv[0, pl.ds(H + c * 16, 16)] = maxv[pl.ds(c * 16, 16)]
            pltpu.sync_copy(outv, r_hbm.at[pl.ds(g, 1)])

    return _readout


def _mm_body_plain(x_ref, w_ref, bf_ref, h_ref, gam_ref, bet_ref, out_ref):
    _mm_core(x_ref[...], w_ref, bf_ref, h_ref, gam_ref, bet_ref, out_ref)


def _mm_body_elu(x_ref, w_ref, bf_ref, h_ref, gam_ref, bet_ref, out_ref):
    xb = x_ref[...]
    xb = jnp.where(xb > 0.0, xb, jnp.exp(jnp.minimum(xb, 0.0)) - 1.0)
    _mm_core(xb, w_ref, bf_ref, h_ref, gam_ref, bet_ref, out_ref)


def _mm_body_bn(x_ref, w_ref, bf_ref, mv_ref, g_ref, b_ref,
                h_ref, gam_ref, bet_ref, out_ref):
    xb = x_ref[...]
    mean = mv_ref[0, :]
    var = mv_ref[1, :]
    xn = (xb - mean[None, :]) * lax.rsqrt(var + 1e-5)[None, :]
    xn = xn * g_ref[0, :][None, :] + b_ref[0, :][None, :]
    xn = jnp.where(xn > 0.0, xn, jnp.exp(jnp.minimum(xn, 0.0)) - 1.0)
    rows = lax.broadcasted_iota(jnp.int32, xb.shape, 0) + pl.program_id(0) * RB
    xb = jnp.where(rows < N, xn, 0.0)
    _mm_core(xb, w_ref, bf_ref, h_ref, gam_ref, bet_ref, out_ref)


def _mm_core(xb, w_ref, bf_ref, h_ref, gam_ref, bet_ref, out_ref):
    y = jnp.dot(xb, w_ref[...], preferred_element_type=jnp.float32)
    h_ref[...] = y[:, :H]
    film = y[:, H:3 * H] + bf_ref[0, :][None, :]
    bet_ref[...] = film[:, :H]
    gam_ref[...] = film[:, H:]
    out_ref[...] = jnp.maximum(
        y[:, 5 * H:6 * H] * y[:, 3 * H:4 * H] + y[:, 4 * H:5 * H], 0.0)


def _layer_mm(xin, wcat, bfilm, bn=None):
    din = xin.shape[1]
    outs = [jax.ShapeDtypeStruct((NP, H), jnp.float32)] * 4
    full = lambda a: pl.BlockSpec(a.shape, lambda i: (0,) * a.ndim)
    in_specs = [
        pl.BlockSpec((RB, din), lambda i: (i, 0)),
        full(wcat),
        full(bfilm),
    ]
    args = [xin, wcat, bfilm]
    if bn is None:
        body = _mm_body_plain
    elif bn == "elu":
        body = _mm_body_elu
    else:
        body = _mm_body_bn
        mv, gg, gb = bn
        in_specs += [full(mv), full(gg), full(gb)]
        args += [mv, gg, gb]
    return pl.pallas_call(
        body,
        grid=(NP // RB,),
        in_specs=in_specs,
        out_specs=[pl.BlockSpec((RB, H), lambda i: (i, 0))] * 4,
        out_shape=outs,
    )(*args)


def _red_body(y_ref, mv_ref):
    i = pl.program_id(0)

    @pl.when(i == 0)
    def _():
        mv_ref[...] = jnp.zeros_like(mv_ref)

    yb = y_ref[...]
    mv_ref[0, :] += jnp.sum(yb, axis=0)
    mv_ref[1, :] += jnp.sum(yb * yb, axis=0)

    @pl.when(i == NRED - 1)
    def _():
        m = mv_ref[0, :] * (1.0 / N)
        v = mv_ref[1, :] * (1.0 / N) - m * m
        mv_ref[0, :] = m
        mv_ref[1, :] = v


def _col_meanvar(y):
    return pl.pallas_call(
        _red_body,
        grid=(NRED,),
        in_specs=[pl.BlockSpec((RRB, H), lambda i: (i, 0))],
        out_specs=pl.BlockSpec((8, H), lambda i: (0, 0)),
        out_shape=jax.ShapeDtypeStruct((8, H), jnp.float32),
    )(y)


def _head_body(r_ref, w1_ref, b1_ref, g_ref, b_ref, w2_ref, b2_ref, o_ref):
    z = jnp.dot(r_ref[...], w1_ref[...], preferred_element_type=jnp.float32)
    z = z + b1_ref[0, :][None, :]
    m = jnp.mean(z, axis=0)
    v = jnp.mean(z * z, axis=0) - m * m
    zn = (z - m[None, :]) * lax.rsqrt(v + 1e-5)[None, :]
    zn = zn * g_ref[0, :][None, :] + b_ref[0, :][None, :]
    ze = jnp.where(zn > 0.0, zn, jnp.exp(jnp.minimum(zn, 0.0)) - 1.0)
    o_ref[...] = jnp.dot(ze, w2_ref[...],
                         preferred_element_type=jnp.float32) + b2_ref[0, :][None, :]


def _head(r, w1, b1, g, b, w2p, b2p):
    full = lambda a: pl.BlockSpec(a.shape, lambda: (0,) * a.ndim)
    return pl.pallas_call(
        _head_body,
        in_specs=[full(r), full(w1), full(b1), full(g), full(b),
                  full(w2p), full(b2p)],
        out_specs=full(jax.ShapeDtypeStruct((G, 128), jnp.float32)),
        out_shape=jax.ShapeDtypeStruct((G, 128), jnp.float32),
    )(r, w1, b1, g, b, w2p, b2p)


def kernel(x, edge_index, edge_attr, batch,
           W1_lin, W1_film, b1_film, W1_skip, W1_fs,
           W2_lin, W2_film, b2_film, W2_skip, W2_fs,
           W3_lin, W3_film, b3_film, W3_skip, W3_fs,
           gbn_g, gbn_b, bn_g, bn_b, lin1_W, lin1_b, cls_W, cls_b):
    src, dst = edge_index[0], edge_index[1]

    packed = lax.sort(dst * 16384 + src)
    src_s = packed & 16383
    row_off = jnp.searchsorted(
        packed, jnp.arange(NP + 1, dtype=jnp.int32) * 16384).astype(jnp.int32)
    deg = row_off[1:] - row_off[:-1]
    off_p = jnp.pad(row_off, (0, (NW - 1) * NPW + OFFW - (NP + 1)),
                    mode="edge")
    deg_p = jnp.pad(deg, (0, (NW - 1) * NPW + OFFW - NP))
    invdeg_p = 1.0 / jnp.maximum(deg_p.astype(jnp.float32), 1.0)
    src_pad = jnp.pad(src_s, (0, WIN + 512))
    b_off = jnp.searchsorted(
        batch, jnp.arange(G + 1, dtype=jnp.int32)).astype(jnp.int32)
    b_off_p = jnp.pad(b_off, (0, 80 - (G + 1)), mode="edge")
    binv_p = 1.0 / jnp.maximum(
        (b_off_p[1:] - b_off_p[:-1]).astype(jnp.float32), 1.0)
    binv_p = jnp.pad(binv_p, (0, 80 - binv_p.shape[0]))

    xp = jnp.pad(x, ((0, NP - N), (0, 0)))
    wcat1 = jnp.concatenate([W1_lin, W1_film, W1_skip, W1_fs], axis=1)
    wcat2 = jnp.concatenate([W2_lin, W2_film, W2_skip, W2_fs], axis=1)
    wcat3 = jnp.concatenate([W3_lin, W3_film, W3_skip, W3_fs], axis=1)
    bf1 = b1_film.reshape(1, 2 * H)
    bf2 = b2_film.reshape(1, 2 * H)
    bf3 = b3_film.reshape(1, 2 * H)

    h, gam, bet, out = _layer_mm(xp, wcat1, bf1)
    y1 = _make_agg()(h, gam, bet, out, off_p, invdeg_p, src_pad)
    mv = _col_meanvar(y1)

    h, gam, bet, out = _layer_mm(
        y1, wcat2, bf2, bn=(mv, gbn_g.reshape(1, H), gbn_b.reshape(1, H)))
    x2 = _make_agg()(h, gam, bet, out, off_p, invdeg_p, src_pad)

    h, gam, bet, out = _layer_mm(x2, wcat3, bf3, bn="elu")
    x3 = _make_agg()(h, gam, bet, out, off_p, invdeg_p, src_pad)

    r = _make_readout()(x3, b_off_p, binv_p)
    w2p = jnp.pad(cls_W, ((0, 0), (0, 128 - NC)))
    b2p = jnp.pad(cls_b, (0, 128 - NC)).reshape(1, 128)
    logits = _head(r, lin1_W, lin1_b.reshape(1, H),
                   bn_g.reshape(1, H), bn_b.reshape(1, H), w2p, b2p)
    return logits[:, :NC]
